# Initial kernel scaffold; baseline (speedup 1.0000x reference)
import jax
import jax.numpy as jnp
from jax.experimental import pallas as pl


def kernel(features, edge_index, adj_values, W1, b1, W2, b2):
    raise NotImplementedError("write your pallas kernel here")


# speedup vs baseline: 4.8929x; 4.8929x over previous
import functools

import jax
import jax.numpy as jnp
from jax import lax
from jax.experimental import pallas as pl
from jax.experimental.pallas import tpu as pltpu
from jax.experimental.pallas import tpu_sc as plsc

_N = 10000
_E = 320000
_D = 128
_H = 32
_O = 64

_NC = 2
_NS = 16
_NW = _NC * _NS
_K = 128
_CHUNKS = 80
_EPT = _K * _CHUNKS
_EPAD = _NW * _EPT
_NP = 10240
_RPT = _NP // _NS
_ZROWS = 128


def _mlp_body(f_ref, w1_ref, b1_ref, w2_ref, b2_ref, o_ref):
    h = jnp.dot(f_ref[...], w1_ref[...], preferred_element_type=jnp.float32)
    h = jnp.maximum(h + b1_ref[...], 0.0)
    o_ref[...] = (
        jnp.dot(h, w2_ref[...], preferred_element_type=jnp.float32) + b2_ref[...]
    )


def _mlp(features, W1, b1, W2, b2):
    blk = 1000
    grid = _N // blk
    return pl.pallas_call(
        _mlp_body,
        grid=(grid,),
        in_specs=[
            pl.BlockSpec((blk, _D), lambda i: (i, 0)),
            pl.BlockSpec((_D, _H), lambda i: (0, 0)),
            pl.BlockSpec((1, _H), lambda i: (0, 0)),
            pl.BlockSpec((_H, _O), lambda i: (0, 0)),
            pl.BlockSpec((1, _O), lambda i: (0, 0)),
        ],
        out_specs=pl.BlockSpec((blk, _O), lambda i: (i, 0)),
        out_shape=jax.ShapeDtypeStruct((_N, _O), jnp.float32),
    )(features, W1, b1, W2, b2)


_mesh = plsc.VectorSubcoreMesh(core_axis_name="c", subcore_axis_name="s")


def _bcast_lane(v, l):
    return lax.gather(
        v,
        jnp.full((16, 1), l, jnp.int32),
        dimension_numbers=lax.GatherDimensionNumbers(
            offset_dims=(), collapsed_slice_dims=(0,), start_index_map=(0,)
        ),
        slice_sizes=(1,),
        mode=lax.GatherScatterMode.PROMISE_IN_BOUNDS,
    )


@functools.partial(
    pl.kernel,
    out_type=jax.ShapeDtypeStruct((_NC, _NP, _O), jnp.float32),
    compiler_params=pltpu.CompilerParams(use_tc_tiling_on_sc=False),
    mesh=_mesh,
    scratch_types=[
        pltpu.VMEM_SHARED((_NP, _O), jnp.float32),
        pltpu.VMEM((_EPT,), jnp.int32),
        pltpu.VMEM((_CHUNKS, _K), jnp.int32),
        pltpu.VMEM((_EPT,), jnp.float32),
        pltpu.VMEM((_K, _O), jnp.float32),
        pltpu.VMEM((_K, _O), jnp.float32),
        pltpu.VMEM((_ZROWS, _O), jnp.float32),
        pltpu.SemaphoreType.DMA,
        pltpu.SemaphoreType.DMA,
    ],
)
def _aggregate(x_hbm, src_hbm, dst_hbm, adj_hbm, out_hbm,
               acc, src_v, dst_v, adj_v, rows0, rows1, zbuf, sem0, sem1):
    cid = lax.axis_index("c")
    sid = lax.axis_index("s")
    wid = cid * _NS + sid

    @pl.loop(0, _ZROWS)
    def _zero_zbuf(r):
        for c in range(_O // 16):
            zbuf[r, pl.ds(c * 16, 16)] = jnp.zeros((16,), jnp.float32)

    base = sid * _RPT
    for j in range(_RPT // _ZROWS):
        pltpu.sync_copy(zbuf, acc.at[pl.ds(base + j * _ZROWS, _ZROWS)])
    plsc.subcore_barrier()

    pltpu.sync_copy(src_hbm.at[wid], src_v)
    pltpu.sync_copy(dst_hbm.at[wid], dst_v)
    pltpu.sync_copy(adj_hbm.at[wid], adj_v)

    def _gather_start(i, buf, sem):
        pltpu.async_copy(x_hbm.at[src_v.at[pl.ds(i * _K, _K)]], buf, sem)

    def _gather_wait(i, buf, sem):
        pltpu.make_async_copy(
            x_hbm.at[src_v.at[pl.ds(i * _K, _K)]], buf, sem
        ).wait()

    def _scale(i, buf):
        @pl.loop(0, _K // 16)
        def _s(g):
            agrp = adj_v[pl.ds(i * _K + g * 16, 16)]
            for l in range(16):
                av = _bcast_lane(agrp, l)
                e = g * 16 + l
                for c in range(_O // 16):
                    sl = pl.ds(c * 16, 16)
                    buf[e, sl] = buf[e, sl] * av

    def _scatter(i, buf):
        pltpu.sync_copy(buf, acc.at[dst_v.at[i]], add=True)

    _gather_start(0, rows0, sem0)

    @pl.loop(0, _CHUNKS, step=2)
    def _main(g):
        _gather_wait(g, rows0, sem0)
        _gather_start(g + 1, rows1, sem1)
        _scale(g, rows0)
        _scatter(g, rows0)

        _gather_wait(g + 1, rows1, sem1)

        @pl.when(g + 2 < _CHUNKS)
        def _():
            _gather_start(g + 2, rows0, sem0)

        _scale(g + 1, rows1)
        _scatter(g + 1, rows1)

    plsc.subcore_barrier()
    pltpu.sync_copy(
        acc.at[pl.ds(base, _RPT)],
        out_hbm.at[cid, pl.ds(base, _RPT)],
    )


def _add_body(a_ref, b_ref, o_ref):
    o_ref[...] = a_ref[0] + b_ref[0]


def _combine(partials):
    blk = 1000
    grid = _N // blk
    return pl.pallas_call(
        _add_body,
        grid=(grid,),
        in_specs=[
            pl.BlockSpec((1, blk, _O), lambda i: (0, i, 0)),
            pl.BlockSpec((1, blk, _O), lambda i: (1, i, 0)),
        ],
        out_specs=pl.BlockSpec((blk, _O), lambda i: (i, 0)),
        out_shape=jax.ShapeDtypeStruct((_N, _O), jnp.float32),
    )(partials, partials)


def kernel(features, edge_index, adj_values, W1, b1, W2, b2):
    x = _mlp(features, W1, b1.reshape(1, _H), W2, b2.reshape(1, _O))

    pad = _EPAD - _E
    src = jnp.pad(edge_index[1], (0, pad)).reshape(_NW, _EPT)
    dst = jnp.pad(edge_index[0], (0, pad)).reshape(_NW, _CHUNKS, _K)
    adj = jnp.pad(adj_values, (0, pad)).reshape(_NW, _EPT)

    partials = _aggregate(x, src, dst, adj)
    return _combine(partials)


# speedup vs baseline: 5.3170x; 1.0867x over previous
import functools

import jax
import jax.numpy as jnp
from jax import lax
from jax.experimental import pallas as pl
from jax.experimental.pallas import tpu as pltpu
from jax.experimental.pallas import tpu_sc as plsc

_N = 10000
_E = 320000
_D = 128
_H = 32
_O = 64

_NC = 2
_NS = 16
_NW = _NC * _NS
_K = 128
_CHUNKS = 80
_EPT = _K * _CHUNKS
_EPAD = _NW * _EPT
_NP = 10240
_RPT = _NP // _NS
_ZROWS = 128


def _mlp_body(f_ref, w1_ref, b1_ref, w2_ref, b2_ref, o_ref):
    h = jnp.dot(f_ref[...], w1_ref[...], preferred_element_type=jnp.float32)
    h = jnp.maximum(h + b1_ref[...], 0.0)
    o_ref[...] = (
        jnp.dot(h, w2_ref[...], preferred_element_type=jnp.float32) + b2_ref[...]
    )


def _mlp(features, W1, b1, W2, b2):
    blk = 1000
    grid = _N // blk
    return pl.pallas_call(
        _mlp_body,
        grid=(grid,),
        in_specs=[
            pl.BlockSpec((blk, _D), lambda i: (i, 0)),
            pl.BlockSpec((_D, _H), lambda i: (0, 0)),
            pl.BlockSpec((1, _H), lambda i: (0, 0)),
            pl.BlockSpec((_H, _O), lambda i: (0, 0)),
            pl.BlockSpec((1, _O), lambda i: (0, 0)),
        ],
        out_specs=pl.BlockSpec((blk, _O), lambda i: (i, 0)),
        out_shape=jax.ShapeDtypeStruct((_N, _O), jnp.float32),
    )(features, W1, b1, W2, b2)


_mesh = plsc.VectorSubcoreMesh(core_axis_name="c", subcore_axis_name="s")


def _bcast_lane(v, l):
    return lax.gather(
        v,
        jnp.full((16, 1), l, jnp.int32),
        dimension_numbers=lax.GatherDimensionNumbers(
            offset_dims=(), collapsed_slice_dims=(0,), start_index_map=(0,)
        ),
        slice_sizes=(1,),
        mode=lax.GatherScatterMode.PROMISE_IN_BOUNDS,
    )


@functools.partial(
    pl.kernel,
    out_type=jax.ShapeDtypeStruct((_NC, _NP, _O), jnp.float32),
    compiler_params=pltpu.CompilerParams(use_tc_tiling_on_sc=False),
    mesh=_mesh,
    scratch_types=[
        pltpu.VMEM_SHARED((_NP, _O), jnp.float32),
        pltpu.VMEM((_EPT,), jnp.int32),
        pltpu.VMEM((_CHUNKS, _K), jnp.int32),
        pltpu.VMEM((_EPT,), jnp.float32),
        pltpu.VMEM((_K, _O), jnp.float32),
        pltpu.VMEM((_K, _O), jnp.float32),
        pltpu.VMEM((_K, _O), jnp.float32),
        pltpu.VMEM((_K, _O), jnp.float32),
        pltpu.VMEM((_ZROWS, _O), jnp.float32),
        pltpu.SemaphoreType.DMA,
        pltpu.SemaphoreType.DMA,
        pltpu.SemaphoreType.DMA,
        pltpu.SemaphoreType.DMA,
    ],
)
def _aggregate(x_hbm, src_hbm, dst_hbm, adj_hbm, out_hbm,
               acc, src_v, dst_v, adj_v, rows0, rows1, srows0, srows1, zbuf,
               sem0, sem1, ssem0, ssem1):
    cid = lax.axis_index("c")
    sid = lax.axis_index("s")
    wid = cid * _NS + sid

    @pl.loop(0, _ZROWS)
    def _zero_zbuf(r):
        for c in range(_O // 16):
            zbuf[r, pl.ds(c * 16, 16)] = jnp.zeros((16,), jnp.float32)

    base = sid * _RPT
    for j in range(_RPT // _ZROWS):
        pltpu.sync_copy(zbuf, acc.at[pl.ds(base + j * _ZROWS, _ZROWS)])
    plsc.subcore_barrier()

    pltpu.sync_copy(src_hbm.at[wid], src_v)
    pltpu.sync_copy(dst_hbm.at[wid], dst_v)
    pltpu.sync_copy(adj_hbm.at[wid], adj_v)

    def _gather_start(i, buf, sem):
        pltpu.async_copy(x_hbm.at[src_v.at[pl.ds(i * _K, _K)]], buf, sem)

    def _gather_wait(i, buf, sem):
        pltpu.make_async_copy(
            x_hbm.at[src_v.at[pl.ds(i * _K, _K)]], buf, sem
        ).wait()

    def _scale(i, buf, sbuf):
        @pl.loop(0, _K // 16)
        def _s(g):
            agrp = adj_v[pl.ds(i * _K + g * 16, 16)]
            for l in range(16):
                av = _bcast_lane(agrp, l)
                e = g * 16 + l
                for c in range(_O // 16):
                    sl = pl.ds(c * 16, 16)
                    sbuf[e, sl] = buf[e, sl] * av

    def _scatter_start(i, sbuf, ssem):
        pltpu.async_copy(sbuf, acc.at[dst_v.at[i]], ssem, add=True)

    def _scatter_wait(i, sbuf, ssem):
        pltpu.make_async_copy(sbuf, acc.at[dst_v.at[i]], ssem).wait()

    _gather_start(0, rows0, sem0)

    @pl.loop(0, _CHUNKS, step=2)
    def _main(g):
        _gather_wait(g, rows0, sem0)
        _gather_start(g + 1, rows1, sem1)

        @pl.when(g >= 2)
        def _():
            _scatter_wait(g - 2, srows0, ssem0)

        _scale(g, rows0, srows0)
        _scatter_start(g, srows0, ssem0)

        _gather_wait(g + 1, rows1, sem1)

        @pl.when(g + 2 < _CHUNKS)
        def _():
            _gather_start(g + 2, rows0, sem0)

        @pl.when(g >= 2)
        def _():
            _scatter_wait(g - 1, srows1, ssem1)

        _scale(g + 1, rows1, srows1)
        _scatter_start(g + 1, srows1, ssem1)

    _scatter_wait(_CHUNKS - 2, srows0, ssem0)
    _scatter_wait(_CHUNKS - 1, srows1, ssem1)

    plsc.subcore_barrier()
    pltpu.sync_copy(
        acc.at[pl.ds(base, _RPT)],
        out_hbm.at[cid, pl.ds(base, _RPT)],
    )


def _add_body(a_ref, b_ref, o_ref):
    o_ref[...] = a_ref[0] + b_ref[0]


def _combine(partials):
    blk = 1000
    grid = _N // blk
    return pl.pallas_call(
        _add_body,
        grid=(grid,),
        in_specs=[
            pl.BlockSpec((1, blk, _O), lambda i: (0, i, 0)),
            pl.BlockSpec((1, blk, _O), lambda i: (1, i, 0)),
        ],
        out_specs=pl.BlockSpec((blk, _O), lambda i: (i, 0)),
        out_shape=jax.ShapeDtypeStruct((_N, _O), jnp.float32),
    )(partials, partials)


def kernel(features, edge_index, adj_values, W1, b1, W2, b2):
    x = _mlp(features, W1, b1.reshape(1, _H), W2, b2.reshape(1, _O))

    pad = _EPAD - _E
    src = jnp.pad(edge_index[1], (0, pad)).reshape(_NW, _EPT)
    dst = jnp.pad(edge_index[0], (0, pad)).reshape(_NW, _CHUNKS, _K)
    adj = jnp.pad(adj_values, (0, pad)).reshape(_NW, _EPT)

    partials = _aggregate(x, src, dst, adj)
    return _combine(partials)


# speedup vs baseline: 5.6704x; 1.0665x over previous
import functools

import jax
import jax.numpy as jnp
from jax import lax
from jax.experimental import pallas as pl
from jax.experimental.pallas import tpu as pltpu
from jax.experimental.pallas import tpu_sc as plsc

_N = 10000
_E = 320000
_D = 128
_H = 32
_O = 64

_NC = 2
_NS = 16
_NW = _NC * _NS
_K = 128
_CHUNKS = 80
_EPT = _K * _CHUNKS
_EPAD = _NW * _EPT
_NP = 10240
_RPT = _NP // _NS
_ZROWS = 128
_GB = 4


def _mlp_body(f_ref, w1_ref, b1_ref, w2_ref, b2_ref, o_ref):
    h = jnp.dot(f_ref[...], w1_ref[...], preferred_element_type=jnp.float32)
    h = jnp.maximum(h + b1_ref[...], 0.0)
    o_ref[...] = (
        jnp.dot(h, w2_ref[...], preferred_element_type=jnp.float32) + b2_ref[...]
    )


def _mlp(features, W1, b1, W2, b2):
    blk = 1000
    grid = _N // blk
    return pl.pallas_call(
        _mlp_body,
        grid=(grid,),
        in_specs=[
            pl.BlockSpec((blk, _D), lambda i: (i, 0)),
            pl.BlockSpec((_D, _H), lambda i: (0, 0)),
            pl.BlockSpec((1, _H), lambda i: (0, 0)),
            pl.BlockSpec((_H, _O), lambda i: (0, 0)),
            pl.BlockSpec((1, _O), lambda i: (0, 0)),
        ],
        out_specs=pl.BlockSpec((blk, _O), lambda i: (i, 0)),
        out_shape=jax.ShapeDtypeStruct((_N, _O), jnp.float32),
    )(features, W1, b1, W2, b2)


_mesh = plsc.VectorSubcoreMesh(core_axis_name="c", subcore_axis_name="s")


def _bcast_lane(v, l):
    return lax.gather(
        v,
        jnp.full((16, 1), l, jnp.int32),
        dimension_numbers=lax.GatherDimensionNumbers(
            offset_dims=(), collapsed_slice_dims=(0,), start_index_map=(0,)
        ),
        slice_sizes=(1,),
        mode=lax.GatherScatterMode.PROMISE_IN_BOUNDS,
    )


@functools.partial(
    pl.kernel,
    out_type=jax.ShapeDtypeStruct((_NC, _NP, _O), jnp.float32),
    compiler_params=pltpu.CompilerParams(use_tc_tiling_on_sc=False),
    mesh=_mesh,
    scratch_types=[
        pltpu.VMEM_SHARED((_NP, _O), jnp.float32),
        pltpu.VMEM((_EPT,), jnp.int32),
        pltpu.VMEM((_CHUNKS, _K), jnp.int32),
        pltpu.VMEM((_EPT,), jnp.float32),
        [pltpu.VMEM((_K, _O), jnp.float32)] * _GB,
        [pltpu.VMEM((_K, _O), jnp.float32)] * 2,
        pltpu.VMEM((_ZROWS, _O), jnp.float32),
        [pltpu.SemaphoreType.DMA] * _GB,
        [pltpu.SemaphoreType.DMA] * 2,
    ],
)
def _aggregate(x_hbm, src_hbm, dst_hbm, adj_hbm, out_hbm,
               acc, src_v, dst_v, adj_v, rows, srows, zbuf, gsems, ssems):
    cid = lax.axis_index("c")
    sid = lax.axis_index("s")
    wid = cid * _NS + sid

    @pl.loop(0, _ZROWS)
    def _zero_zbuf(r):
        for c in range(_O // 16):
            zbuf[r, pl.ds(c * 16, 16)] = jnp.zeros((16,), jnp.float32)

    base = sid * _RPT
    for j in range(_RPT // _ZROWS):
        pltpu.sync_copy(zbuf, acc.at[pl.ds(base + j * _ZROWS, _ZROWS)])
    plsc.subcore_barrier()

    pltpu.sync_copy(src_hbm.at[wid], src_v)
    pltpu.sync_copy(dst_hbm.at[wid], dst_v)
    pltpu.sync_copy(adj_hbm.at[wid], adj_v)

    def _gather_start(i, buf, sem):
        pltpu.async_copy(x_hbm.at[src_v.at[pl.ds(i * _K, _K)]], buf, sem)

    def _gather_wait(i, buf, sem):
        pltpu.make_async_copy(
            x_hbm.at[src_v.at[pl.ds(i * _K, _K)]], buf, sem
        ).wait()

    def _scale(i, buf, sbuf):
        @pl.loop(0, _K // 16)
        def _s(g):
            agrp = adj_v[pl.ds(i * _K + g * 16, 16)]
            for l in range(16):
                av = _bcast_lane(agrp, l)
                e = g * 16 + l
                for c in range(_O // 16):
                    sl = pl.ds(c * 16, 16)
                    sbuf[e, sl] = buf[e, sl] * av

    def _scatter_start(i, sbuf, ssem):
        pltpu.async_copy(sbuf, acc.at[dst_v.at[i]], ssem, add=True)

    def _scatter_wait(i, sbuf, ssem):
        pltpu.make_async_copy(sbuf, acc.at[dst_v.at[i]], ssem).wait()

    for b in range(_GB):
        _gather_start(b, rows[b], gsems[b])

    @pl.loop(0, _CHUNKS, step=_GB)
    def _main(g):
        for b in range(_GB):
            i = g + b
            sb = b % 2
            _gather_wait(i, rows[b], gsems[b])

            @pl.when(i >= 2)
            def _():
                _scatter_wait(i - 2, srows[sb], ssems[sb])

            _scale(i, rows[b], srows[sb])
            _scatter_start(i, srows[sb], ssems[sb])

            @pl.when(i + _GB < _CHUNKS)
            def _():
                _gather_start(i + _GB, rows[b], gsems[b])

    _scatter_wait(_CHUNKS - 2, srows[0], ssems[0])
    _scatter_wait(_CHUNKS - 1, srows[1], ssems[1])

    plsc.subcore_barrier()
    pltpu.sync_copy(
        acc.at[pl.ds(base, _RPT)],
        out_hbm.at[cid, pl.ds(base, _RPT)],
    )


def _add_body(a_ref, b_ref, o_ref):
    o_ref[...] = a_ref[0] + b_ref[0]


def _combine(partials):
    blk = 1000
    grid = _N // blk
    return pl.pallas_call(
        _add_body,
        grid=(grid,),
        in_specs=[
            pl.BlockSpec((1, blk, _O), lambda i: (0, i, 0)),
            pl.BlockSpec((1, blk, _O), lambda i: (1, i, 0)),
        ],
        out_specs=pl.BlockSpec((blk, _O), lambda i: (i, 0)),
        out_shape=jax.ShapeDtypeStruct((_N, _O), jnp.float32),
    )(partials, partials)


def kernel(features, edge_index, adj_values, W1, b1, W2, b2):
    x = _mlp(features, W1, b1.reshape(1, _H), W2, b2.reshape(1, _O))

    pad = _EPAD - _E
    src = jnp.pad(edge_index[1], (0, pad)).reshape(_NW, _EPT)
    dst = jnp.pad(edge_index[0], (0, pad)).reshape(_NW, _CHUNKS, _K)
    adj = jnp.pad(adj_values, (0, pad)).reshape(_NW, _EPT)

    partials = _aggregate(x, src, dst, adj)
    return _combine(partials)


# speedup vs baseline: 5.6769x; 1.0011x over previous
import functools

import jax
import jax.numpy as jnp
from jax import lax
from jax.experimental import pallas as pl
from jax.experimental.pallas import tpu as pltpu
from jax.experimental.pallas import tpu_sc as plsc

_N = 10000
_E = 320000
_D = 128
_H = 32
_O = 64

_NC = 2
_NS = 16
_NW = _NC * _NS
_K = 128
_CHUNKS = 80
_EPT = _K * _CHUNKS
_EPAD = _NW * _EPT
_NP = 10240
_RPT = _NP // _NS
_ZROWS = 128
_GB = 4


def _mlp_body(f_ref, w1_ref, b1_ref, w2_ref, b2_ref, o_ref):
    h = jnp.dot(f_ref[...], w1_ref[...], preferred_element_type=jnp.float32)
    h = jnp.maximum(h + b1_ref[...], 0.0)
    o_ref[...] = (
        jnp.dot(h, w2_ref[...], preferred_element_type=jnp.float32) + b2_ref[...]
    )


def _mlp(features, W1, b1, W2, b2):
    blk = 1000
    grid = _N // blk
    return pl.pallas_call(
        _mlp_body,
        grid=(grid,),
        in_specs=[
            pl.BlockSpec((blk, _D), lambda i: (i, 0)),
            pl.BlockSpec((_D, _H), lambda i: (0, 0)),
            pl.BlockSpec((1, _H), lambda i: (0, 0)),
            pl.BlockSpec((_H, _O), lambda i: (0, 0)),
            pl.BlockSpec((1, _O), lambda i: (0, 0)),
        ],
        out_specs=pl.BlockSpec((blk, _O), lambda i: (i, 0)),
        out_shape=jax.ShapeDtypeStruct((_N, _O), jnp.float32),
    )(features, W1, b1, W2, b2)


_mesh = plsc.VectorSubcoreMesh(core_axis_name="c", subcore_axis_name="s")


def _bcast_lane(v, l):
    return lax.gather(
        v,
        jnp.full((16, 1), l, jnp.int32),
        dimension_numbers=lax.GatherDimensionNumbers(
            offset_dims=(), collapsed_slice_dims=(0,), start_index_map=(0,)
        ),
        slice_sizes=(1,),
        mode=lax.GatherScatterMode.PROMISE_IN_BOUNDS,
    )


@functools.partial(
    pl.kernel,
    out_type=jax.ShapeDtypeStruct((_NC, _NP, _O), jnp.float32),
    compiler_params=pltpu.CompilerParams(use_tc_tiling_on_sc=False),
    mesh=_mesh,
    scratch_types=[
        pltpu.VMEM_SHARED((_NP, _O), jnp.float32),
        pltpu.VMEM((_EPT,), jnp.int32),
        pltpu.VMEM((_CHUNKS, _K), jnp.int32),
        pltpu.VMEM((_EPT,), jnp.float32),
        [pltpu.VMEM((_K, _O), jnp.float32)] * _GB,
        [pltpu.VMEM((_K, _O), jnp.float32)] * 2,
        pltpu.VMEM((_ZROWS, _O), jnp.float32),
        [pltpu.SemaphoreType.DMA] * _GB,
        [pltpu.SemaphoreType.DMA] * 2,
    ],
)
def _aggregate(x_hbm, src_hbm, dst_hbm, adj_hbm, out_hbm,
               acc, src_v, dst_v, adj_v, rows, srows, zbuf, gsems, ssems):
    cid = lax.axis_index("c")
    sid = lax.axis_index("s")
    wid = cid * _NS + sid

    @pl.loop(0, _ZROWS)
    def _zero_zbuf(r):
        for c in range(_O // 16):
            zbuf[r, pl.ds(c * 16, 16)] = jnp.zeros((16,), jnp.float32)

    base = sid * _RPT
    for j in range(_RPT // _ZROWS):
        pltpu.sync_copy(zbuf, acc.at[pl.ds(base + j * _ZROWS, _ZROWS)])
    plsc.subcore_barrier()

    pltpu.sync_copy(src_hbm.at[wid], src_v)
    pltpu.sync_copy(dst_hbm.at[wid], dst_v)
    pltpu.sync_copy(adj_hbm.at[wid], adj_v)

    def _gather_start(i, buf, sem):
        pltpu.async_copy(x_hbm.at[src_v.at[pl.ds(i * _K, _K)]], buf, sem)

    def _gather_wait(i, buf, sem):
        pltpu.make_async_copy(
            x_hbm.at[src_v.at[pl.ds(i * _K, _K)]], buf, sem
        ).wait()

    def _scale(i, buf, sbuf):
        @plsc.parallel_loop(0, _K // 16, unroll=2)
        def _s(g):
            agrp = adj_v[pl.ds(i * _K + g * 16, 16)]
            for l in range(16):
                av = _bcast_lane(agrp, l)
                e = g * 16 + l
                for c in range(_O // 16):
                    sl = pl.ds(c * 16, 16)
                    sbuf[e, sl] = buf[e, sl] * av

    def _scatter_start(i, sbuf, ssem):
        pltpu.async_copy(sbuf, acc.at[dst_v.at[i]], ssem, add=True)

    def _scatter_wait(i, sbuf, ssem):
        pltpu.make_async_copy(sbuf, acc.at[dst_v.at[i]], ssem).wait()

    for b in range(_GB):
        _gather_start(b, rows[b], gsems[b])

    @pl.loop(0, _CHUNKS, step=_GB)
    def _main(g):
        for b in range(_GB):
            i = g + b
            sb = b % 2
            _gather_wait(i, rows[b], gsems[b])

            @pl.when(i >= 2)
            def _():
                _scatter_wait(i - 2, srows[sb], ssems[sb])

            _scale(i, rows[b], srows[sb])
            _scatter_start(i, srows[sb], ssems[sb])

            @pl.when(i + _GB < _CHUNKS)
            def _():
                _gather_start(i + _GB, rows[b], gsems[b])

    _scatter_wait(_CHUNKS - 2, srows[0], ssems[0])
    _scatter_wait(_CHUNKS - 1, srows[1], ssems[1])

    plsc.subcore_barrier()
    pltpu.sync_copy(
        acc.at[pl.ds(base, _RPT)],
        out_hbm.at[cid, pl.ds(base, _RPT)],
    )


def _add_body(a_ref, b_ref, o_ref):
    o_ref[...] = a_ref[0] + b_ref[0]


def _combine(partials):
    blk = 1000
    grid = _N // blk
    return pl.pallas_call(
        _add_body,
        grid=(grid,),
        in_specs=[
            pl.BlockSpec((1, blk, _O), lambda i: (0, i, 0)),
            pl.BlockSpec((1, blk, _O), lambda i: (1, i, 0)),
        ],
        out_specs=pl.BlockSpec((blk, _O), lambda i: (i, 0)),
        out_shape=jax.ShapeDtypeStruct((_N, _O), jnp.float32),
    )(partials, partials)


def kernel(features, edge_index, adj_values, W1, b1, W2, b2):
    x = _mlp(features, W1, b1.reshape(1, _H), W2, b2.reshape(1, _O))

    pad = _EPAD - _E
    src = jnp.pad(edge_index[1], (0, pad)).reshape(_NW, _EPT)
    dst = jnp.pad(edge_index[0], (0, pad)).reshape(_NW, _CHUNKS, _K)
    adj = jnp.pad(adj_values, (0, pad)).reshape(_NW, _EPT)

    partials = _aggregate(x, src, dst, adj)
    return _combine(partials)


# speedup vs baseline: 8.1500x; 1.4357x over previous
import functools

import jax
import jax.numpy as jnp
from jax import lax
from jax.experimental import pallas as pl
from jax.experimental.pallas import tpu as pltpu
from jax.experimental.pallas import tpu_sc as plsc

_N = 10000
_E = 320000
_D = 128
_H = 32
_O = 64

_NC = 2
_NS = 16
_NW = _NC * _NS
_C = _O // _NW
_K = 2048
_CHUNKS = 157
_EPAD = _K * _CHUNKS


def _mlp_body(f_ref, w1_ref, b1_ref, w2_ref, b2_ref, o_ref):
    h = jnp.dot(f_ref[...], w1_ref[...], preferred_element_type=jnp.float32)
    h = jnp.maximum(h + b1_ref[...], 0.0)
    y = jnp.dot(h, w2_ref[...], preferred_element_type=jnp.float32) + b2_ref[...]
    o_ref[...] = y.T


def _mlp_t(features, W1, b1, W2, b2):
    return pl.pallas_call(
        _mlp_body,
        out_shape=jax.ShapeDtypeStruct((_O, _N), jnp.float32),
    )(features, W1, b1, W2, b2)


_mesh = plsc.VectorSubcoreMesh(core_axis_name="c", subcore_axis_name="s")


@functools.partial(
    pl.kernel,
    out_type=jax.ShapeDtypeStruct((_O, _N), jnp.float32),
    compiler_params=pltpu.CompilerParams(
        use_tc_tiling_on_sc=False, needs_layout_passes=False
    ),
    mesh=_mesh,
    scratch_types=[
        pltpu.VMEM((_C, _N), jnp.float32),
        pltpu.VMEM((_C, _N), jnp.float32),
        [pltpu.VMEM((_K,), jnp.int32)] * 2,
        [pltpu.VMEM((_K,), jnp.int32)] * 2,
        [pltpu.VMEM((_K,), jnp.float32)] * 2,
        [pltpu.SemaphoreType.DMA] * 2,
    ],
)
def _aggregate(xt_hbm, src_hbm, dst_hbm, adj_hbm, out_hbm,
               xt_v, acc_v, srcs, dsts, adjs, sems):
    cid = lax.axis_index("c")
    sid = lax.axis_index("s")
    wid = cid * _NS + sid
    colbase = wid * _C

    pltpu.sync_copy(xt_hbm.at[pl.ds(colbase, _C)], xt_v)

    @pl.loop(0, _N // 16)
    def _zero(r):
        for c in range(_C):
            acc_v[c, pl.ds(r * 16, 16)] = jnp.zeros((16,), jnp.float32)

    def _load_start(i, b):
        off = pl.ds(i * _K, _K)
        pltpu.async_copy(src_hbm.at[off], srcs[b], sems[b])
        pltpu.async_copy(dst_hbm.at[off], dsts[b], sems[b])
        pltpu.async_copy(adj_hbm.at[off], adjs[b], sems[b])

    def _load_wait(i, b):
        off = pl.ds(i * _K, _K)
        pltpu.make_async_copy(src_hbm.at[off], srcs[b], sems[b]).wait()
        pltpu.make_async_copy(dst_hbm.at[off], dsts[b], sems[b]).wait()
        pltpu.make_async_copy(adj_hbm.at[off], adjs[b], sems[b]).wait()

    def _process(b):
        @plsc.parallel_loop(0, _K // 16, unroll=4)
        def _grp(g):
            sl = pl.ds(g * 16, 16)
            sv = srcs[b][sl]
            dv = dsts[b][sl]
            av = adjs[b][sl]
            for c in range(_C):
                cv = jnp.full((16,), c, jnp.int32)
                x = plsc.load_gather(xt_v, [cv, sv])
                plsc.addupdate_scatter(acc_v, [cv, dv], x * av)

    _load_start(0, 0)

    @pl.loop(0, _CHUNKS - 1, step=2)
    def _main(i):
        _load_wait(i, 0)
        _load_start(i + 1, 1)
        _process(0)
        _load_wait(i + 1, 1)

        @pl.when(i + 2 < _CHUNKS)
        def _():
            _load_start(i + 2, 0)

        _process(1)

    _load_wait(_CHUNKS - 1, 0)
    _process(0)

    pltpu.sync_copy(acc_v, out_hbm.at[pl.ds(colbase, _C)])


def _t_body(a_ref, o_ref):
    o_ref[...] = a_ref[...].T


def _transpose_back(aggT):
    return pl.pallas_call(
        _t_body,
        out_shape=jax.ShapeDtypeStruct((_N, _O), jnp.float32),
    )(aggT)


def kernel(features, edge_index, adj_values, W1, b1, W2, b2):
    xT = _mlp_t(features, W1, b1.reshape(1, _H), W2, b2.reshape(1, _O))

    pad = _EPAD - _E
    src = jnp.pad(edge_index[1], (0, pad))
    dst = jnp.pad(edge_index[0], (0, pad))
    adj = jnp.pad(adj_values, (0, pad))

    aggT = _aggregate(xT, src, dst, adj)
    return _transpose_back(aggT)
